# baseline (speedup 1.0000x reference)
import functools

import numpy as np
import jax
import jax.numpy as jnp
from jax.experimental import pallas as pl
from jax.experimental.pallas import tpu as pltpu

D_MODEL = 768
N_HEADS = 12
D_HEAD = 64
D_FF = 2048
FACTOR = 5
NUM_CLASS = 10

_LAYER_L = (2049, 1025, 513)
_LAYER_LP = (2304, 1280, 768)
_RB = 256

_NEG = -1e30
_INV_SQRT2 = 0.7071067811865476


def _gelu(x):
    return 0.5 * x * (1.0 + jax.lax.erf(x * _INV_SQRT2))


def _build_constants():
    root = jax.random.key(123)
    counts, masks, us = [], [], []
    for li, L in enumerate(_LAYER_L):
        u = min(FACTOR * int(np.ceil(np.log(L))), L)
        idx = np.asarray(
            jax.random.randint(jax.random.fold_in(root, li), (L, u), 0, L))
        Lp = _LAYER_LP[li]
        cnt = np.zeros((Lp, Lp), np.float32)
        np.add.at(cnt, (np.repeat(np.arange(L), u), idx.reshape(-1)), 1.0)
        msk = np.where(cnt > 0.0, 0.0, _NEG).astype(jnp.bfloat16)
        counts.append(cnt.astype(jnp.bfloat16))
        masks.append(msk)
        us.append(u)
    return counts, masks, us


_COUNTS, _MASKS, _US = _build_constants()


def _pos_emb_np(L, d):
    pos = np.arange(L, dtype=np.float32)[:, None]
    div = np.exp(np.arange(0, d, 2, dtype=np.float32) * -(np.log(10000.0) / d))
    pe = np.zeros((L, d), dtype=np.float32)
    pe[:, 0::2] = np.sin(pos * div)
    pe[:, 1::2] = np.cos(pos * div)
    return pe


_POS = _pos_emb_np(_LAYER_L[0], D_MODEL)


def _bdot(a, b):
    return jnp.dot(a.astype(jnp.bfloat16), b,
                   preferred_element_type=jnp.float32)


def _mm_kernel(x_ref, w_ref, b_ref, o_ref):
    o_ref[...] = (_bdot(x_ref[...], w_ref[...])
                  + b_ref[...]).astype(o_ref.dtype)


def _mm_call(x, w, b, out_dtype=jnp.float32):
    R, C = x.shape
    C2 = w.shape[1]
    nb = -(-R // _RB)
    return pl.pallas_call(
        _mm_kernel,
        grid=(nb,),
        in_specs=[pl.BlockSpec((_RB, C), lambda i: (i, 0)),
                  pl.BlockSpec((C, C2), lambda i: (0, 0)),
                  pl.BlockSpec((1, C2), lambda i: (0, 0))],
        out_specs=pl.BlockSpec((_RB, C2), lambda i: (i, 0)),
        out_shape=jax.ShapeDtypeStruct((R, C2), out_dtype),
    )(x, w, b)


def _token_kernel(xm_ref, x0_ref, xp_ref, w0_ref, w1_ref, w2_ref, pos_ref,
                  o_ref):
    acc = _bdot(xm_ref[0], w0_ref[...])
    acc += _bdot(x0_ref[0], w1_ref[...])
    acc += _bdot(xp_ref[0], w2_ref[...])
    o_ref[0] = acc + pos_ref[...]


def _token_call(xm, x0, xp, w0, w1, w2, pos):
    B, L, C = x0.shape
    nb = -(-L // _RB)
    xspec = pl.BlockSpec((1, _RB, C), lambda b, i: (b, i, 0))
    wspec = pl.BlockSpec((C, D_MODEL), lambda b, i: (0, 0))
    return pl.pallas_call(
        _token_kernel,
        grid=(B, nb),
        in_specs=[xspec, xspec, xspec, wspec, wspec, wspec,
                  pl.BlockSpec((_RB, D_MODEL), lambda b, i: (i, 0))],
        out_specs=pl.BlockSpec((1, _RB, D_MODEL), lambda b, i: (b, i, 0)),
        out_shape=jax.ShapeDtypeStruct((B, L, D_MODEL), jnp.float32),
    )(xm, x0, xp, w0, w1, w2, pos)


def _conv_kernel(xm_ref, x0_ref, xp_ref, w0_ref, w1_ref, w2_ref, s_ref,
                 b_ref, o_ref):
    acc = _bdot(xm_ref[...], w0_ref[...])
    acc += _bdot(x0_ref[...], w1_ref[...])
    acc += _bdot(xp_ref[...], w2_ref[...])
    h = acc * s_ref[...] + b_ref[...]
    o_ref[...] = jnp.where(h > 0.0, h, jnp.exp(h) - 1.0)


def _conv_call(xm, x0, xp, w0, w1, w2, scale, bias):
    R, C = x0.shape
    nb = -(-R // _RB)
    xspec = pl.BlockSpec((_RB, C), lambda i: (i, 0))
    wspec = pl.BlockSpec((C, C), lambda i: (0, 0))
    vspec = pl.BlockSpec((1, C), lambda i: (0, 0))
    return pl.pallas_call(
        _conv_kernel,
        grid=(nb,),
        in_specs=[xspec, xspec, xspec, wspec, wspec, wspec, vspec, vspec],
        out_specs=xspec,
        out_shape=jax.ShapeDtypeStruct((R, C), jnp.float32),
    )(xm, x0, xp, w0, w1, w2, scale, bias)


def _pool_kernel(a_ref, b_ref, c_ref, o_ref):
    o_ref[...] = jnp.maximum(jnp.maximum(a_ref[...], b_ref[...]), c_ref[...])


def _pool_call(a, b, c):
    R, C = a.shape
    nb = -(-R // _RB)
    spec = pl.BlockSpec((_RB, C), lambda i: (i, 0))
    return pl.pallas_call(
        _pool_kernel,
        grid=(nb,),
        in_specs=[spec, spec, spec],
        out_specs=spec,
        out_shape=jax.ShapeDtypeStruct((R, C), jnp.float32),
    )(a, b, c)


def _m_kernel(q_ref, kt_ref, kh_ref, c_ref, msk_ref, m_ref, *, L):
    s = jnp.dot(q_ref[0], kt_ref[0], preferred_element_type=jnp.float32)
    mx = jnp.max(s + msk_ref[...], axis=1, keepdims=True)
    ksum = jnp.dot(c_ref[...], kh_ref[0],
                   preferred_element_type=jnp.float32)
    sm = jnp.sum(q_ref[0].astype(jnp.float32) * ksum, axis=1,
                 keepdims=True) * (1.0 / L)
    m_ref[0, 0] = mx - sm


def _m_call(qh, kt, kh, cnt, msk, L, Lp):
    BH = qh.shape[0]
    nq = Lp // _RB
    m4 = pl.pallas_call(
        functools.partial(_m_kernel, L=L),
        grid=(nq, BH),
        in_specs=[
            pl.BlockSpec((1, _RB, D_HEAD), lambda i, j: (j, i, 0)),
            pl.BlockSpec((1, D_HEAD, Lp), lambda i, j: (j, 0, 0)),
            pl.BlockSpec((1, Lp, D_HEAD), lambda i, j: (j, 0, 0)),
            pl.BlockSpec((_RB, Lp), lambda i, j: (i, 0)),
            pl.BlockSpec((_RB, Lp), lambda i, j: (i, 0)),
        ],
        out_specs=pl.BlockSpec((1, 1, _RB, 1), lambda i, j: (i, j, 0, 0)),
        out_shape=jax.ShapeDtypeStruct((nq, BH, _RB, 1), jnp.float32),
    )(qh, kt, kh, cnt, msk)
    return jnp.transpose(m4, (1, 0, 2, 3)).reshape(BH, Lp)


def _select_kernel(m_ref, o_ref, *, Lp, u):
    iota = jax.lax.broadcasted_iota(jnp.int32, (32, Lp), 1)
    cols = jax.lax.broadcasted_iota(jnp.int32, (32, 64), 1)

    def pick(j, carry):
        m, acc = carry
        mx = jnp.max(m, axis=1, keepdims=True)
        am = jnp.min(jnp.where(m == mx, iota, Lp), axis=1, keepdims=True)
        acc = jnp.where(cols == j, am.astype(jnp.float32), acc)
        return jnp.where(iota == am, _NEG, m), acc

    _, idxs = jax.lax.fori_loop(
        0, u, pick, (m_ref[...], jnp.full((32, 64), float(Lp), jnp.float32)))
    o_ref[...] = idxs


def _select_call(m, Lp, u):
    return pl.pallas_call(
        functools.partial(_select_kernel, Lp=Lp, u=u),
        grid=(1,),
        in_specs=[pl.BlockSpec((32, Lp), lambda i: (0, 0))],
        out_specs=pl.BlockSpec((32, 64), lambda i: (0, 0)),
        out_shape=jax.ShapeDtypeStruct((32, 64), jnp.float32),
    )(m)


def _attn_kernel(ir_ref, ic_ref, q_ref, kt_ref, v_ref, o_ref, *, L, Lp):
    v = v_ref[0]
    vmean = jnp.sum(v.astype(jnp.float32), axis=0, keepdims=True) * (1.0 / L)

    iota_r = jax.lax.broadcasted_iota(jnp.int32, (1, Lp), 1)
    iota_c = jax.lax.broadcasted_iota(jnp.int32, (Lp, 1), 0)
    ic = ic_ref[0].astype(jnp.int32)
    ir = ir_ref[0].astype(jnp.int32)
    onehot = (ic == iota_r).astype(jnp.bfloat16)
    onehot_tb = iota_c == ir

    qr = jnp.dot(onehot, q_ref[0], preferred_element_type=jnp.float32)
    s = _bdot(qr, kt_ref[0]) * 0.125
    s = jnp.where(iota_r < L, s, _NEG)
    s = s - jnp.max(s, axis=1, keepdims=True)
    e = jnp.exp(s)
    p = e / jnp.sum(e, axis=1, keepdims=True)
    upd = _bdot(p, v)

    selmask = jnp.sum(onehot_tb.astype(jnp.float32), axis=1, keepdims=True)
    ctx = (jnp.dot(onehot_tb.astype(jnp.bfloat16), upd.astype(jnp.bfloat16),
                   preferred_element_type=jnp.float32)
           + (1.0 - selmask) * vmean)
    o_ref[0] = ctx.astype(o_ref.dtype)


def _attn_call(idx, qh, kt, vh, L, Lp):
    BH = qh.shape[0]
    ir = idx[:BH].reshape(BH, 1, 64)
    ic = idx[:BH].reshape(BH, 64, 1)
    return pl.pallas_call(
        functools.partial(_attn_kernel, L=L, Lp=Lp),
        grid=(BH,),
        in_specs=[
            pl.BlockSpec((1, 1, 64), lambda j: (j, 0, 0)),
            pl.BlockSpec((1, 64, 1), lambda j: (j, 0, 0)),
            pl.BlockSpec((1, Lp, D_HEAD), lambda j: (j, 0, 0)),
            pl.BlockSpec((1, D_HEAD, Lp), lambda j: (j, 0, 0)),
            pl.BlockSpec((1, Lp, D_HEAD), lambda j: (j, 0, 0)),
        ],
        out_specs=pl.BlockSpec((1, Lp, D_HEAD), lambda j: (j, 0, 0)),
        out_shape=jax.ShapeDtypeStruct((BH, Lp, D_HEAD), jnp.bfloat16),
    )(ir, ic, qh, kt, vh)


def _ln(h, g, b):
    mu = jnp.mean(h, axis=1, keepdims=True)
    d = h - mu
    var = jnp.mean(d * d, axis=1, keepdims=True)
    return d / jnp.sqrt(var + 1e-5) * g + b


def _post_kernel(x_ref, a_ref, wo_ref, bo_ref, g1_ref, b1_ref, w1_ref,
                 bf1_ref, w2_ref, bf2_ref, g2_ref, b2_ref, o_ref):
    att = _bdot(a_ref[...], wo_ref[...]) + bo_ref[...]
    h = _ln(x_ref[...] + att, g1_ref[...], b1_ref[...])
    f = _bdot(h, w1_ref[...]) + bf1_ref[...]
    f = _gelu(f)
    y = _bdot(f, w2_ref[...]) + bf2_ref[...]
    o_ref[...] = _ln(h + y, g2_ref[...], b2_ref[...])


def _post_call(x, a, wo, bo, g1, b1, w1, bf1, w2, bf2, g2, b2):
    R = x.shape[0]
    nb = -(-R // _RB)
    xspec = pl.BlockSpec((_RB, D_MODEL), lambda i: (i, 0))
    vspec = pl.BlockSpec((1, D_MODEL), lambda i: (0, 0))
    fspec = pl.BlockSpec((1, D_FF), lambda i: (0, 0))
    return pl.pallas_call(
        _post_kernel,
        grid=(nb,),
        in_specs=[xspec, xspec,
                  pl.BlockSpec((D_MODEL, D_MODEL), lambda i: (0, 0)), vspec,
                  vspec, vspec,
                  pl.BlockSpec((D_MODEL, D_FF), lambda i: (0, 0)), fspec,
                  pl.BlockSpec((D_FF, D_MODEL), lambda i: (0, 0)), vspec,
                  vspec, vspec],
        out_specs=xspec,
        out_shape=jax.ShapeDtypeStruct((R, D_MODEL), jnp.float32),
    )(x, a, wo, bo, g1, b1, w1, bf1, w2, bf2, g2, b2)


def _final_kernel(x_ref, g_ref, b_ref, o_ref):
    h = _ln(x_ref[...], g_ref[...], b_ref[...])
    o_ref[...] = _gelu(h)


def _final_call(x, g, b):
    R = x.shape[0]
    nb = -(-R // _RB)
    xspec = pl.BlockSpec((_RB, D_MODEL), lambda i: (i, 0))
    vspec = pl.BlockSpec((1, D_MODEL), lambda i: (0, 0))
    return pl.pallas_call(
        _final_kernel,
        grid=(nb,),
        in_specs=[xspec, vspec, vspec],
        out_specs=xspec,
        out_shape=jax.ShapeDtypeStruct((R, D_MODEL), jnp.float32),
    )(x, g, b)


def _proj_kernel(z_ref, pw_ref, pb_ref, o_ref):
    @pl.when(pl.program_id(0) == 0)
    def _():
        o_ref[...] = jnp.broadcast_to(pb_ref[...], o_ref.shape)

    o_ref[...] += _bdot(z_ref[...], pw_ref[...])


def _proj_call(z, pw, pb):
    B, K = z.shape
    kb = K // 9
    return pl.pallas_call(
        _proj_kernel,
        grid=(9,),
        in_specs=[pl.BlockSpec((B, kb), lambda i: (0, i)),
                  pl.BlockSpec((kb, NUM_CLASS), lambda i: (i, 0)),
                  pl.BlockSpec((1, NUM_CLASS), lambda i: (0, 0))],
        out_specs=pl.BlockSpec((B, NUM_CLASS), lambda i: (0, 0)),
        out_shape=jax.ShapeDtypeStruct((B, NUM_CLASS), jnp.float32),
    )(z, pw, pb)


def kernel(x_enc, params):
    B, L0, _ = x_enc.shape

    bf16 = jnp.bfloat16
    tw = params['token_w'].astype(bf16)
    xm = jnp.roll(x_enc, 1, axis=1)
    xp = jnp.roll(x_enc, -1, axis=1)
    x = _token_call(xm, x_enc, xp, tw[:, :, 0].T, tw[:, :, 1].T,
                    tw[:, :, 2].T, jnp.asarray(_POS))
    x = x.reshape(B * L0, D_MODEL)

    L = L0
    for li in range(3):
        Lp = _LAYER_LP[li]
        u = _US[li]
        p = params['layers'][li]

        wqkv = jnp.concatenate([p['Wq'].T, p['Wk'].T,
                                p['Wv'].T], axis=1).astype(bf16)
        bqkv = jnp.concatenate([p['bq'], p['bk'], p['bv']])[None, :]
        qkv = _mm_call(x, wqkv, bqkv,
                       out_dtype=bf16).reshape(B, L, 3, N_HEADS, D_HEAD)

        def prep(t):
            t = jnp.transpose(t, (0, 2, 1, 3)).reshape(B * N_HEADS, L, D_HEAD)
            return jnp.pad(t, ((0, 0), (0, Lp - L), (0, 0)))

        qh = prep(qkv[:, :, 0])
        kh = prep(qkv[:, :, 1])
        vh = prep(qkv[:, :, 2])
        kt = jnp.transpose(kh, (0, 2, 1))

        m = _m_call(qh, kt, kh, jnp.asarray(_COUNTS[li]),
                    jnp.asarray(_MASKS[li]), L, Lp)
        m32 = jnp.pad(m, ((0, 32 - m.shape[0]), (0, 0)),
                      constant_values=_NEG)
        idx = _select_call(m32, Lp, u)
        ctx = _attn_call(idx, qh, kt, vh, L, Lp)
        a = (ctx[:, :L, :].reshape(B, N_HEADS, L, D_HEAD)
             .transpose(0, 2, 1, 3).reshape(B * L, D_MODEL))

        x = _post_call(x, a, p['Wo'].T.astype(bf16), p['bo'][None],
                       p['g1'][None], p['b1'][None], p['w1'].T.astype(bf16),
                       p['bf1'][None], p['w2'].T.astype(bf16),
                       p['bf2'][None], p['g2'][None], p['b2'][None])

        if li < 2:
            c = params['convs'][li]
            xb = x.reshape(B, L, D_MODEL)
            am1 = jnp.roll(xb, 1, 1).reshape(B * L, D_MODEL)
            ap1 = jnp.roll(xb, -1, 1).reshape(B * L, D_MODEL)
            scale = (c['bg'] / np.sqrt(1.0 + 1e-5))[None]
            bias = (c['cb'] * scale[0] + c['bb'])[None]
            cw = c['cw'].astype(bf16)
            h = _conv_call(am1, x, ap1, cw[:, :, 0].T, cw[:, :, 1].T,
                           cw[:, :, 2].T, scale, bias)
            hb = h.reshape(B, L, D_MODEL)
            hp = jnp.pad(hb, ((0, 0), (1, 1), (0, 0)),
                         constant_values=-jnp.inf)
            P = (L - 1) // 2 + 1
            ca = hp[:, 0::2][:, :P].reshape(B * P, D_MODEL)
            aa = hp[:, 1::2][:, :P].reshape(B * P, D_MODEL)
            ba = hp[:, 2::2][:, :P].reshape(B * P, D_MODEL)
            x = _pool_call(aa, ba, ca)
            L = P

    z = _final_call(x, params['norm_g'][None], params['norm_b'][None])
    zf = z.reshape(B, L * D_MODEL)
    return _proj_call(zf, params['proj_w'].T.astype(bf16),
                      params['proj_b'][None])


# speedup vs baseline: 1.0433x; 1.0433x over previous
import functools

import numpy as np
import jax
import jax.numpy as jnp
from jax.experimental import pallas as pl
from jax.experimental.pallas import tpu as pltpu

D_MODEL = 768
N_HEADS = 12
D_HEAD = 64
D_FF = 2048
FACTOR = 5
NUM_CLASS = 10

_LAYER_L = (2049, 1025, 513)
_LAYER_LP = (2304, 1280, 768)
_RB = 256

_NEG = -1e30
_INV_SQRT2 = 0.7071067811865476


def _gelu(x):
    return 0.5 * x * (1.0 + jax.lax.erf(x * _INV_SQRT2))


def _build_constants():
    root = jax.random.key(123)
    counts, masks, us = [], [], []
    for li, L in enumerate(_LAYER_L):
        u = min(FACTOR * int(np.ceil(np.log(L))), L)
        idx = np.asarray(
            jax.random.randint(jax.random.fold_in(root, li), (L, u), 0, L))
        Lp = _LAYER_LP[li]
        cnt = np.zeros((Lp, Lp), np.float32)
        np.add.at(cnt, (np.repeat(np.arange(L), u), idx.reshape(-1)), 1.0)
        msk = np.where(cnt > 0.0, 0.0, _NEG).astype(jnp.bfloat16)
        counts.append(cnt.astype(jnp.bfloat16))
        masks.append(msk)
        us.append(u)
    return counts, masks, us


_COUNTS, _MASKS, _US = _build_constants()


def _pos_emb_np(L, d):
    pos = np.arange(L, dtype=np.float32)[:, None]
    div = np.exp(np.arange(0, d, 2, dtype=np.float32) * -(np.log(10000.0) / d))
    pe = np.zeros((L, d), dtype=np.float32)
    pe[:, 0::2] = np.sin(pos * div)
    pe[:, 1::2] = np.cos(pos * div)
    return pe


_POS = _pos_emb_np(_LAYER_L[0], D_MODEL)


def _bdot(a, b):
    return jnp.dot(a.astype(jnp.bfloat16), b,
                   preferred_element_type=jnp.float32)


def _mm_kernel(x_ref, w_ref, b_ref, o_ref):
    o_ref[...] = (_bdot(x_ref[...], w_ref[...])
                  + b_ref[...]).astype(o_ref.dtype)


def _mm_call(x, w, b, out_dtype=jnp.float32):
    R, C = x.shape
    C2 = w.shape[1]
    nb = -(-R // _RB)
    return pl.pallas_call(
        _mm_kernel,
        grid=(nb,),
        in_specs=[pl.BlockSpec((_RB, C), lambda i: (i, 0)),
                  pl.BlockSpec((C, C2), lambda i: (0, 0)),
                  pl.BlockSpec((1, C2), lambda i: (0, 0))],
        out_specs=pl.BlockSpec((_RB, C2), lambda i: (i, 0)),
        out_shape=jax.ShapeDtypeStruct((R, C2), out_dtype),
    )(x, w, b)


def _token_kernel(xm_ref, x0_ref, xp_ref, w0_ref, w1_ref, w2_ref, pos_ref,
                  o_ref):
    acc = _bdot(xm_ref[0], w0_ref[...])
    acc += _bdot(x0_ref[0], w1_ref[...])
    acc += _bdot(xp_ref[0], w2_ref[...])
    o_ref[0] = acc + pos_ref[...]


def _token_call(xm, x0, xp, w0, w1, w2, pos):
    B, L, C = x0.shape
    nb = -(-L // _RB)
    xspec = pl.BlockSpec((1, _RB, C), lambda b, i: (b, i, 0))
    wspec = pl.BlockSpec((C, D_MODEL), lambda b, i: (0, 0))
    return pl.pallas_call(
        _token_kernel,
        grid=(B, nb),
        in_specs=[xspec, xspec, xspec, wspec, wspec, wspec,
                  pl.BlockSpec((_RB, D_MODEL), lambda b, i: (i, 0))],
        out_specs=pl.BlockSpec((1, _RB, D_MODEL), lambda b, i: (b, i, 0)),
        out_shape=jax.ShapeDtypeStruct((B, L, D_MODEL), jnp.float32),
    )(xm, x0, xp, w0, w1, w2, pos)


def _conv_kernel(xm_ref, x0_ref, xp_ref, w0_ref, w1_ref, w2_ref, s_ref,
                 b_ref, o_ref):
    acc = _bdot(xm_ref[...], w0_ref[...])
    acc += _bdot(x0_ref[...], w1_ref[...])
    acc += _bdot(xp_ref[...], w2_ref[...])
    h = acc * s_ref[...] + b_ref[...]
    o_ref[...] = jnp.where(h > 0.0, h, jnp.exp(h) - 1.0)


def _conv_call(xm, x0, xp, w0, w1, w2, scale, bias):
    R, C = x0.shape
    nb = -(-R // _RB)
    xspec = pl.BlockSpec((_RB, C), lambda i: (i, 0))
    wspec = pl.BlockSpec((C, C), lambda i: (0, 0))
    vspec = pl.BlockSpec((1, C), lambda i: (0, 0))
    return pl.pallas_call(
        _conv_kernel,
        grid=(nb,),
        in_specs=[xspec, xspec, xspec, wspec, wspec, wspec, vspec, vspec],
        out_specs=xspec,
        out_shape=jax.ShapeDtypeStruct((R, C), jnp.float32),
    )(xm, x0, xp, w0, w1, w2, scale, bias)


def _pool_kernel(a_ref, b_ref, c_ref, o_ref):
    o_ref[...] = jnp.maximum(jnp.maximum(a_ref[...], b_ref[...]), c_ref[...])


def _pool_call(a, b, c):
    R, C = a.shape
    nb = -(-R // _RB)
    spec = pl.BlockSpec((_RB, C), lambda i: (i, 0))
    return pl.pallas_call(
        _pool_kernel,
        grid=(nb,),
        in_specs=[spec, spec, spec],
        out_specs=spec,
        out_shape=jax.ShapeDtypeStruct((R, C), jnp.float32),
    )(a, b, c)


def _m_kernel(q_ref, kt_ref, c_ref, msk_ref, m_ref, *, L):
    s = jnp.dot(q_ref[0], kt_ref[0], preferred_element_type=jnp.float32)
    mx = jnp.max(s + msk_ref[...], axis=1, keepdims=True)
    sm = jnp.sum(s * c_ref[...], axis=1, keepdims=True) * (1.0 / L)
    m_ref[0, 0] = mx - sm


def _m_call(qh, kt, cnt, msk, L, Lp):
    BH = qh.shape[0]
    nq = Lp // _RB
    m4 = pl.pallas_call(
        functools.partial(_m_kernel, L=L),
        grid=(nq, BH),
        in_specs=[
            pl.BlockSpec((1, _RB, D_HEAD), lambda i, j: (j, i, 0)),
            pl.BlockSpec((1, D_HEAD, Lp), lambda i, j: (j, 0, 0)),
            pl.BlockSpec((_RB, Lp), lambda i, j: (i, 0)),
            pl.BlockSpec((_RB, Lp), lambda i, j: (i, 0)),
        ],
        out_specs=pl.BlockSpec((1, 1, _RB, 1), lambda i, j: (i, j, 0, 0)),
        out_shape=jax.ShapeDtypeStruct((nq, BH, _RB, 1), jnp.float32),
    )(qh, kt, cnt, msk)
    return jnp.transpose(m4, (1, 0, 2, 3)).reshape(BH, Lp)


def _select_kernel(m_ref, o_ref, *, Lp, u):
    iota = jax.lax.broadcasted_iota(jnp.int32, (32, Lp), 1)
    cols = jax.lax.broadcasted_iota(jnp.int32, (32, 64), 1)

    def pick(j, carry):
        m, acc = carry
        mx = jnp.max(m, axis=1, keepdims=True)
        am = jnp.min(jnp.where(m == mx, iota, Lp), axis=1, keepdims=True)
        acc = jnp.where(cols == j, am.astype(jnp.float32), acc)
        return jnp.where(iota == am, _NEG, m), acc

    _, idxs = jax.lax.fori_loop(
        0, u, pick, (m_ref[...], jnp.full((32, 64), float(Lp), jnp.float32)))
    o_ref[...] = idxs


def _select_call(m, Lp, u):
    return pl.pallas_call(
        functools.partial(_select_kernel, Lp=Lp, u=u),
        grid=(1,),
        in_specs=[pl.BlockSpec((32, Lp), lambda i: (0, 0))],
        out_specs=pl.BlockSpec((32, 64), lambda i: (0, 0)),
        out_shape=jax.ShapeDtypeStruct((32, 64), jnp.float32),
    )(m)


def _attn_kernel(ir_ref, ic_ref, q_ref, kt_ref, v_ref, o_ref, *, L, Lp):
    v = v_ref[0]
    vmean = jnp.sum(v.astype(jnp.float32), axis=0, keepdims=True) * (1.0 / L)

    iota_r = jax.lax.broadcasted_iota(jnp.int32, (1, Lp), 1)
    iota_c = jax.lax.broadcasted_iota(jnp.int32, (Lp, 1), 0)
    ic = ic_ref[0].astype(jnp.int32)
    ir = ir_ref[0].astype(jnp.int32)
    onehot = (ic == iota_r).astype(jnp.bfloat16)
    onehot_tb = iota_c == ir

    qr = jnp.dot(onehot, q_ref[0], preferred_element_type=jnp.float32)
    s = _bdot(qr, kt_ref[0]) * 0.125
    s = jnp.where(iota_r < L, s, _NEG)
    s = s - jnp.max(s, axis=1, keepdims=True)
    e = jnp.exp(s)
    p = e / jnp.sum(e, axis=1, keepdims=True)
    upd = _bdot(p, v)

    selmask = jnp.sum(onehot_tb.astype(jnp.float32), axis=1, keepdims=True)
    ctx = (jnp.dot(onehot_tb.astype(jnp.bfloat16), upd.astype(jnp.bfloat16),
                   preferred_element_type=jnp.float32)
           + (1.0 - selmask) * vmean)
    o_ref[0] = ctx.astype(o_ref.dtype)


def _attn_call(idx, qh, kt, vh, L, Lp):
    BH = qh.shape[0]
    ir = idx[:BH].reshape(BH, 1, 64)
    ic = idx[:BH].reshape(BH, 64, 1)
    return pl.pallas_call(
        functools.partial(_attn_kernel, L=L, Lp=Lp),
        grid=(BH,),
        in_specs=[
            pl.BlockSpec((1, 1, 64), lambda j: (j, 0, 0)),
            pl.BlockSpec((1, 64, 1), lambda j: (j, 0, 0)),
            pl.BlockSpec((1, Lp, D_HEAD), lambda j: (j, 0, 0)),
            pl.BlockSpec((1, D_HEAD, Lp), lambda j: (j, 0, 0)),
            pl.BlockSpec((1, Lp, D_HEAD), lambda j: (j, 0, 0)),
        ],
        out_specs=pl.BlockSpec((1, Lp, D_HEAD), lambda j: (j, 0, 0)),
        out_shape=jax.ShapeDtypeStruct((BH, Lp, D_HEAD), jnp.bfloat16),
    )(ir, ic, qh, kt, vh)


def _ln(h, g, b):
    mu = jnp.mean(h, axis=1, keepdims=True)
    d = h - mu
    var = jnp.mean(d * d, axis=1, keepdims=True)
    return d / jnp.sqrt(var + 1e-5) * g + b


def _post_kernel(x_ref, a_ref, wo_ref, bo_ref, g1_ref, b1_ref, w1_ref,
                 bf1_ref, w2_ref, bf2_ref, g2_ref, b2_ref, o_ref):
    att = _bdot(a_ref[...], wo_ref[...]) + bo_ref[...]
    h = _ln(x_ref[...] + att, g1_ref[...], b1_ref[...])
    f = _bdot(h, w1_ref[...]) + bf1_ref[...]
    f = _gelu(f)
    y = _bdot(f, w2_ref[...]) + bf2_ref[...]
    o_ref[...] = _ln(h + y, g2_ref[...], b2_ref[...])


def _post_call(x, a, wo, bo, g1, b1, w1, bf1, w2, bf2, g2, b2):
    R = x.shape[0]
    nb = -(-R // _RB)
    xspec = pl.BlockSpec((_RB, D_MODEL), lambda i: (i, 0))
    vspec = pl.BlockSpec((1, D_MODEL), lambda i: (0, 0))
    fspec = pl.BlockSpec((1, D_FF), lambda i: (0, 0))
    return pl.pallas_call(
        _post_kernel,
        grid=(nb,),
        in_specs=[xspec, xspec,
                  pl.BlockSpec((D_MODEL, D_MODEL), lambda i: (0, 0)), vspec,
                  vspec, vspec,
                  pl.BlockSpec((D_MODEL, D_FF), lambda i: (0, 0)), fspec,
                  pl.BlockSpec((D_FF, D_MODEL), lambda i: (0, 0)), vspec,
                  vspec, vspec],
        out_specs=xspec,
        out_shape=jax.ShapeDtypeStruct((R, D_MODEL), jnp.float32),
    )(x, a, wo, bo, g1, b1, w1, bf1, w2, bf2, g2, b2)


def _final_kernel(x_ref, g_ref, b_ref, o_ref):
    h = _ln(x_ref[...], g_ref[...], b_ref[...])
    o_ref[...] = _gelu(h)


def _final_call(x, g, b):
    R = x.shape[0]
    nb = -(-R // _RB)
    xspec = pl.BlockSpec((_RB, D_MODEL), lambda i: (i, 0))
    vspec = pl.BlockSpec((1, D_MODEL), lambda i: (0, 0))
    return pl.pallas_call(
        _final_kernel,
        grid=(nb,),
        in_specs=[xspec, vspec, vspec],
        out_specs=xspec,
        out_shape=jax.ShapeDtypeStruct((R, D_MODEL), jnp.float32),
    )(x, g, b)


def _proj_kernel(z_ref, pw_ref, pb_ref, o_ref):
    @pl.when(pl.program_id(0) == 0)
    def _():
        o_ref[...] = jnp.broadcast_to(pb_ref[...], o_ref.shape)

    o_ref[...] += _bdot(z_ref[...], pw_ref[...])


def _proj_call(z, pw, pb):
    B, K = z.shape
    kb = K // 9
    return pl.pallas_call(
        _proj_kernel,
        grid=(9,),
        in_specs=[pl.BlockSpec((B, kb), lambda i: (0, i)),
                  pl.BlockSpec((kb, NUM_CLASS), lambda i: (i, 0)),
                  pl.BlockSpec((1, NUM_CLASS), lambda i: (0, 0))],
        out_specs=pl.BlockSpec((B, NUM_CLASS), lambda i: (0, 0)),
        out_shape=jax.ShapeDtypeStruct((B, NUM_CLASS), jnp.float32),
    )(z, pw, pb)


def kernel(x_enc, params):
    B, L0, _ = x_enc.shape

    bf16 = jnp.bfloat16
    tw = params['token_w'].astype(bf16)
    xm = jnp.roll(x_enc, 1, axis=1)
    xp = jnp.roll(x_enc, -1, axis=1)
    x = _token_call(xm, x_enc, xp, tw[:, :, 0].T, tw[:, :, 1].T,
                    tw[:, :, 2].T, jnp.asarray(_POS))
    x = x.reshape(B * L0, D_MODEL)

    L = L0
    for li in range(3):
        Lp = _LAYER_LP[li]
        u = _US[li]
        p = params['layers'][li]

        wqkv = jnp.concatenate([p['Wq'].T, p['Wk'].T,
                                p['Wv'].T], axis=1).astype(bf16)
        bqkv = jnp.concatenate([p['bq'], p['bk'], p['bv']])[None, :]
        qkv = _mm_call(x, wqkv, bqkv,
                       out_dtype=bf16).reshape(B, L, 3, N_HEADS, D_HEAD)

        def prep(t):
            t = jnp.transpose(t, (0, 2, 1, 3)).reshape(B * N_HEADS, L, D_HEAD)
            return jnp.pad(t, ((0, 0), (0, Lp - L), (0, 0)))

        qh = prep(qkv[:, :, 0])
        kh = prep(qkv[:, :, 1])
        vh = prep(qkv[:, :, 2])
        kt = jnp.transpose(kh, (0, 2, 1))

        m = _m_call(qh, kt, jnp.asarray(_COUNTS[li]),
                    jnp.asarray(_MASKS[li]), L, Lp)
        m32 = jnp.pad(m, ((0, 32 - m.shape[0]), (0, 0)),
                      constant_values=_NEG)
        idx = _select_call(m32, Lp, u)
        ctx = _attn_call(idx, qh, kt, vh, L, Lp)
        a = (ctx[:, :L, :].reshape(B, N_HEADS, L, D_HEAD)
             .transpose(0, 2, 1, 3).reshape(B * L, D_MODEL))

        x = _post_call(x, a, p['Wo'].T.astype(bf16), p['bo'][None],
                       p['g1'][None], p['b1'][None], p['w1'].T.astype(bf16),
                       p['bf1'][None], p['w2'].T.astype(bf16),
                       p['bf2'][None], p['g2'][None], p['b2'][None])

        if li < 2:
            c = params['convs'][li]
            xb = x.reshape(B, L, D_MODEL)
            am1 = jnp.roll(xb, 1, 1).reshape(B * L, D_MODEL)
            ap1 = jnp.roll(xb, -1, 1).reshape(B * L, D_MODEL)
            scale = (c['bg'] / np.sqrt(1.0 + 1e-5))[None]
            bias = (c['cb'] * scale[0] + c['bb'])[None]
            cw = c['cw'].astype(bf16)
            h = _conv_call(am1, x, ap1, cw[:, :, 0].T, cw[:, :, 1].T,
                           cw[:, :, 2].T, scale, bias)
            hb = h.reshape(B, L, D_MODEL)
            hp = jnp.pad(hb, ((0, 0), (1, 1), (0, 0)),
                         constant_values=-jnp.inf)
            P = (L - 1) // 2 + 1
            ca = hp[:, 0::2][:, :P].reshape(B * P, D_MODEL)
            aa = hp[:, 1::2][:, :P].reshape(B * P, D_MODEL)
            ba = hp[:, 2::2][:, :P].reshape(B * P, D_MODEL)
            x = _pool_call(aa, ba, ca)
            L = P

    z = _final_call(x, params['norm_g'][None], params['norm_b'][None])
    zf = z.reshape(B, L * D_MODEL)
    return _proj_call(zf, params['proj_w'].T.astype(bf16),
                      params['proj_b'][None])


# speedup vs baseline: 1.0521x; 1.0084x over previous
import functools

import numpy as np
import jax
import jax.numpy as jnp
from jax.experimental import pallas as pl
from jax.experimental.pallas import tpu as pltpu

D_MODEL = 768
N_HEADS = 12
D_HEAD = 64
D_FF = 2048
FACTOR = 5
NUM_CLASS = 10

_LAYER_L = (2049, 1025, 513)
_LAYER_LP = (2304, 1280, 768)
_RB = 256

_NEG = -1e30
_INV_SQRT2 = 0.7071067811865476


def _gelu(x):
    return 0.5 * x * (1.0 + jax.lax.erf(x * _INV_SQRT2))


def _build_constants():
    root = jax.random.key(123)
    counts, masks, us = [], [], []
    for li, L in enumerate(_LAYER_L):
        u = min(FACTOR * int(np.ceil(np.log(L))), L)
        idx = np.asarray(
            jax.random.randint(jax.random.fold_in(root, li), (L, u), 0, L))
        Lp = _LAYER_LP[li]
        cnt = np.zeros((Lp, Lp), np.float32)
        np.add.at(cnt, (np.repeat(np.arange(L), u), idx.reshape(-1)), 1.0)
        counts.append(cnt)
        us.append(u)
    return counts, us


_COUNTS, _US = _build_constants()


def _pos_emb_np(L, d):
    pos = np.arange(L, dtype=np.float32)[:, None]
    div = np.exp(np.arange(0, d, 2, dtype=np.float32) * -(np.log(10000.0) / d))
    pe = np.zeros((L, d), dtype=np.float32)
    pe[:, 0::2] = np.sin(pos * div)
    pe[:, 1::2] = np.cos(pos * div)
    return pe


_POS = _pos_emb_np(_LAYER_L[0], D_MODEL)


def _bdot(a, b):
    return jnp.dot(a.astype(jnp.bfloat16), b,
                   preferred_element_type=jnp.float32)


def _mm_kernel(x_ref, w_ref, b_ref, o_ref):
    o_ref[...] = (_bdot(x_ref[...], w_ref[...])
                  + b_ref[...]).astype(o_ref.dtype)


def _mm_call(x, w, b, out_dtype=jnp.float32):
    R, C = x.shape
    C2 = w.shape[1]
    nb = -(-R // _RB)
    return pl.pallas_call(
        _mm_kernel,
        grid=(nb,),
        in_specs=[pl.BlockSpec((_RB, C), lambda i: (i, 0)),
                  pl.BlockSpec((C, C2), lambda i: (0, 0)),
                  pl.BlockSpec((1, C2), lambda i: (0, 0))],
        out_specs=pl.BlockSpec((_RB, C2), lambda i: (i, 0)),
        out_shape=jax.ShapeDtypeStruct((R, C2), out_dtype),
    )(x, w, b)


def _token_kernel(xm_ref, x0_ref, xp_ref, w0_ref, w1_ref, w2_ref, pos_ref,
                  o_ref):
    acc = _bdot(xm_ref[0], w0_ref[...])
    acc += _bdot(x0_ref[0], w1_ref[...])
    acc += _bdot(xp_ref[0], w2_ref[...])
    o_ref[0] = acc + pos_ref[...]


def _token_call(xm, x0, xp, w0, w1, w2, pos):
    B, L, C = x0.shape
    nb = -(-L // _RB)
    xspec = pl.BlockSpec((1, _RB, C), lambda b, i: (b, i, 0))
    wspec = pl.BlockSpec((C, D_MODEL), lambda b, i: (0, 0))
    return pl.pallas_call(
        _token_kernel,
        grid=(B, nb),
        in_specs=[xspec, xspec, xspec, wspec, wspec, wspec,
                  pl.BlockSpec((_RB, D_MODEL), lambda b, i: (i, 0))],
        out_specs=pl.BlockSpec((1, _RB, D_MODEL), lambda b, i: (b, i, 0)),
        out_shape=jax.ShapeDtypeStruct((B, L, D_MODEL), jnp.float32),
    )(xm, x0, xp, w0, w1, w2, pos)


def _conv_kernel(xm_ref, x0_ref, xp_ref, w0_ref, w1_ref, w2_ref, s_ref,
                 b_ref, o_ref):
    acc = _bdot(xm_ref[...], w0_ref[...])
    acc += _bdot(x0_ref[...], w1_ref[...])
    acc += _bdot(xp_ref[...], w2_ref[...])
    h = acc * s_ref[...] + b_ref[...]
    o_ref[...] = jnp.where(h > 0.0, h, jnp.exp(h) - 1.0)


def _conv_call(xm, x0, xp, w0, w1, w2, scale, bias):
    R, C = x0.shape
    nb = -(-R // _RB)
    xspec = pl.BlockSpec((_RB, C), lambda i: (i, 0))
    wspec = pl.BlockSpec((C, C), lambda i: (0, 0))
    vspec = pl.BlockSpec((1, C), lambda i: (0, 0))
    return pl.pallas_call(
        _conv_kernel,
        grid=(nb,),
        in_specs=[xspec, xspec, xspec, wspec, wspec, wspec, vspec, vspec],
        out_specs=xspec,
        out_shape=jax.ShapeDtypeStruct((R, C), jnp.float32),
    )(xm, x0, xp, w0, w1, w2, scale, bias)


def _pool_kernel(a_ref, b_ref, c_ref, o_ref):
    o_ref[...] = jnp.maximum(jnp.maximum(a_ref[...], b_ref[...]), c_ref[...])


def _pool_call(a, b, c):
    R, C = a.shape
    nb = -(-R // _RB)
    spec = pl.BlockSpec((_RB, C), lambda i: (i, 0))
    return pl.pallas_call(
        _pool_kernel,
        grid=(nb,),
        in_specs=[spec, spec, spec],
        out_specs=spec,
        out_shape=jax.ShapeDtypeStruct((R, C), jnp.float32),
    )(a, b, c)


def _m_kernel(q_ref, kt_ref, c_ref, m_ref, *, L):
    s = jnp.dot(q_ref[0], kt_ref[0], preferred_element_type=jnp.float32)
    c = c_ref[...]
    mx = jnp.max(jnp.where(c > 0.0, s, _NEG), axis=1, keepdims=True)
    sm = jnp.sum(s * c, axis=1, keepdims=True) * (1.0 / L)
    m_ref[0, 0] = mx - sm


def _m_call(qh, kt, cnt, L, Lp):
    BH = qh.shape[0]
    nq = Lp // _RB
    m4 = pl.pallas_call(
        functools.partial(_m_kernel, L=L),
        grid=(nq, BH),
        in_specs=[
            pl.BlockSpec((1, _RB, D_HEAD), lambda i, j: (j, i, 0)),
            pl.BlockSpec((1, D_HEAD, Lp), lambda i, j: (j, 0, 0)),
            pl.BlockSpec((_RB, Lp), lambda i, j: (i, 0)),
        ],
        out_specs=pl.BlockSpec((1, 1, _RB, 1), lambda i, j: (i, j, 0, 0)),
        out_shape=jax.ShapeDtypeStruct((nq, BH, _RB, 1), jnp.float32),
    )(qh, kt, cnt)
    return jnp.transpose(m4, (1, 0, 2, 3)).reshape(BH, Lp)


def _select_kernel(m_ref, o_ref, *, Lp, u):
    iota = jax.lax.broadcasted_iota(jnp.int32, (32, Lp), 1)
    cols = jax.lax.broadcasted_iota(jnp.int32, (32, 64), 1)

    def pick(j, carry):
        m, acc = carry
        mx = jnp.max(m, axis=1, keepdims=True)
        am = jnp.min(jnp.where(m == mx, iota, Lp), axis=1, keepdims=True)
        acc = jnp.where(cols == j, am.astype(jnp.float32), acc)
        return jnp.where(iota == am, _NEG, m), acc

    _, idxs = jax.lax.fori_loop(
        0, u, pick, (m_ref[...], jnp.full((32, 64), float(Lp), jnp.float32)))
    o_ref[...] = idxs


def _select_call(m, Lp, u):
    return pl.pallas_call(
        functools.partial(_select_kernel, Lp=Lp, u=u),
        grid=(1,),
        in_specs=[pl.BlockSpec((32, Lp), lambda i: (0, 0))],
        out_specs=pl.BlockSpec((32, 64), lambda i: (0, 0)),
        out_shape=jax.ShapeDtypeStruct((32, 64), jnp.float32),
    )(m)


def _attn_kernel(ir_ref, ic_ref, q_ref, kt_ref, v_ref, o_ref, *, L, Lp):
    v = v_ref[0]
    vmean = jnp.sum(v.astype(jnp.float32), axis=0, keepdims=True) * (1.0 / L)

    iota_r = jax.lax.broadcasted_iota(jnp.int32, (1, Lp), 1)
    iota_c = jax.lax.broadcasted_iota(jnp.int32, (Lp, 1), 0)
    ic = ic_ref[0].astype(jnp.int32)
    ir = ir_ref[0].astype(jnp.int32)
    onehot = (ic == iota_r).astype(jnp.bfloat16)
    onehot_tb = iota_c == ir

    qr = jnp.dot(onehot, q_ref[0], preferred_element_type=jnp.float32)
    s = _bdot(qr, kt_ref[0]) * 0.125
    s = jnp.where(iota_r < L, s, _NEG)
    s = s - jnp.max(s, axis=1, keepdims=True)
    e = jnp.exp(s)
    p = e / jnp.sum(e, axis=1, keepdims=True)
    upd = _bdot(p, v)

    selmask = jnp.sum(onehot_tb.astype(jnp.float32), axis=1, keepdims=True)
    ctx = (jnp.dot(onehot_tb.astype(jnp.bfloat16), upd.astype(jnp.bfloat16),
                   preferred_element_type=jnp.float32)
           + (1.0 - selmask) * vmean)
    o_ref[0] = ctx.astype(o_ref.dtype)


def _attn_call(idx, qh, kt, vh, L, Lp):
    BH = qh.shape[0]
    ir = idx[:BH].reshape(BH, 1, 64)
    ic = idx[:BH].reshape(BH, 64, 1)
    return pl.pallas_call(
        functools.partial(_attn_kernel, L=L, Lp=Lp),
        grid=(BH,),
        in_specs=[
            pl.BlockSpec((1, 1, 64), lambda j: (j, 0, 0)),
            pl.BlockSpec((1, 64, 1), lambda j: (j, 0, 0)),
            pl.BlockSpec((1, Lp, D_HEAD), lambda j: (j, 0, 0)),
            pl.BlockSpec((1, D_HEAD, Lp), lambda j: (j, 0, 0)),
            pl.BlockSpec((1, Lp, D_HEAD), lambda j: (j, 0, 0)),
        ],
        out_specs=pl.BlockSpec((1, Lp, D_HEAD), lambda j: (j, 0, 0)),
        out_shape=jax.ShapeDtypeStruct((BH, Lp, D_HEAD), jnp.bfloat16),
    )(ir, ic, qh, kt, vh)


def _ln(h, g, b):
    mu = jnp.mean(h, axis=1, keepdims=True)
    d = h - mu
    var = jnp.mean(d * d, axis=1, keepdims=True)
    return d / jnp.sqrt(var + 1e-5) * g + b


def _post_kernel(x_ref, a_ref, wo_ref, bo_ref, g1_ref, b1_ref, w1_ref,
                 bf1_ref, w2_ref, bf2_ref, g2_ref, b2_ref, o_ref):
    att = _bdot(a_ref[...], wo_ref[...]) + bo_ref[...]
    h = _ln(x_ref[...] + att, g1_ref[...], b1_ref[...])
    f = _bdot(h, w1_ref[...]) + bf1_ref[...]
    f = _gelu(f)
    y = _bdot(f, w2_ref[...]) + bf2_ref[...]
    o_ref[...] = _ln(h + y, g2_ref[...], b2_ref[...])


def _post_call(x, a, wo, bo, g1, b1, w1, bf1, w2, bf2, g2, b2):
    R = x.shape[0]
    nb = -(-R // _RB)
    xspec = pl.BlockSpec((_RB, D_MODEL), lambda i: (i, 0))
    vspec = pl.BlockSpec((1, D_MODEL), lambda i: (0, 0))
    fspec = pl.BlockSpec((1, D_FF), lambda i: (0, 0))
    return pl.pallas_call(
        _post_kernel,
        grid=(nb,),
        in_specs=[xspec, xspec,
                  pl.BlockSpec((D_MODEL, D_MODEL), lambda i: (0, 0)), vspec,
                  vspec, vspec,
                  pl.BlockSpec((D_MODEL, D_FF), lambda i: (0, 0)), fspec,
                  pl.BlockSpec((D_FF, D_MODEL), lambda i: (0, 0)), vspec,
                  vspec, vspec],
        out_specs=xspec,
        out_shape=jax.ShapeDtypeStruct((R, D_MODEL), jnp.float32),
    )(x, a, wo, bo, g1, b1, w1, bf1, w2, bf2, g2, b2)


def _final_kernel(x_ref, g_ref, b_ref, o_ref):
    h = _ln(x_ref[...], g_ref[...], b_ref[...])
    o_ref[...] = _gelu(h)


def _final_call(x, g, b):
    R = x.shape[0]
    nb = -(-R // _RB)
    xspec = pl.BlockSpec((_RB, D_MODEL), lambda i: (i, 0))
    vspec = pl.BlockSpec((1, D_MODEL), lambda i: (0, 0))
    return pl.pallas_call(
        _final_kernel,
        grid=(nb,),
        in_specs=[xspec, vspec, vspec],
        out_specs=xspec,
        out_shape=jax.ShapeDtypeStruct((R, D_MODEL), jnp.float32),
    )(x, g, b)


def _proj_kernel(z_ref, pw_ref, pb_ref, o_ref):
    @pl.when(pl.program_id(0) == 0)
    def _():
        o_ref[...] = jnp.broadcast_to(pb_ref[...], o_ref.shape)

    o_ref[...] += _bdot(z_ref[...], pw_ref[...])


def _proj_call(z, pw, pb):
    B, K = z.shape
    kb = K // 9
    return pl.pallas_call(
        _proj_kernel,
        grid=(9,),
        in_specs=[pl.BlockSpec((B, kb), lambda i: (0, i)),
                  pl.BlockSpec((kb, NUM_CLASS), lambda i: (i, 0)),
                  pl.BlockSpec((1, NUM_CLASS), lambda i: (0, 0))],
        out_specs=pl.BlockSpec((B, NUM_CLASS), lambda i: (0, 0)),
        out_shape=jax.ShapeDtypeStruct((B, NUM_CLASS), jnp.float32),
    )(z, pw, pb)


def kernel(x_enc, params):
    B, L0, _ = x_enc.shape

    bf16 = jnp.bfloat16
    tw = params['token_w'].astype(bf16)
    xm = jnp.roll(x_enc, 1, axis=1)
    xp = jnp.roll(x_enc, -1, axis=1)
    x = _token_call(xm, x_enc, xp, tw[:, :, 0].T, tw[:, :, 1].T,
                    tw[:, :, 2].T, jnp.asarray(_POS))
    x = x.reshape(B * L0, D_MODEL)

    L = L0
    for li in range(3):
        Lp = _LAYER_LP[li]
        u = _US[li]
        p = params['layers'][li]

        wqkv = jnp.concatenate([p['Wq'].T, p['Wk'].T,
                                p['Wv'].T], axis=1).astype(bf16)
        bqkv = jnp.concatenate([p['bq'], p['bk'], p['bv']])[None, :]
        qkv = _mm_call(x, wqkv, bqkv,
                       out_dtype=bf16).reshape(B, L, 3, N_HEADS, D_HEAD)

        def prep(t):
            t = jnp.transpose(t, (0, 2, 1, 3)).reshape(B * N_HEADS, L, D_HEAD)
            return jnp.pad(t, ((0, 0), (0, Lp - L), (0, 0)))

        qh = prep(qkv[:, :, 0])
        kh = prep(qkv[:, :, 1])
        vh = prep(qkv[:, :, 2])
        kt = jnp.transpose(kh, (0, 2, 1))

        m = _m_call(qh, kt, jnp.asarray(_COUNTS[li]), L, Lp)
        m32 = jnp.pad(m, ((0, 32 - m.shape[0]), (0, 0)),
                      constant_values=_NEG)
        idx = _select_call(m32, Lp, u)
        ctx = _attn_call(idx, qh, kt, vh, L, Lp)
        a = (ctx[:, :L, :].reshape(B, N_HEADS, L, D_HEAD)
             .transpose(0, 2, 1, 3).reshape(B * L, D_MODEL))

        x = _post_call(x, a, p['Wo'].T.astype(bf16), p['bo'][None],
                       p['g1'][None], p['b1'][None], p['w1'].T.astype(bf16),
                       p['bf1'][None], p['w2'].T.astype(bf16),
                       p['bf2'][None], p['g2'][None], p['b2'][None])

        if li < 2:
            c = params['convs'][li]
            xb = x.reshape(B, L, D_MODEL)
            am1 = jnp.roll(xb, 1, 1).reshape(B * L, D_MODEL)
            ap1 = jnp.roll(xb, -1, 1).reshape(B * L, D_MODEL)
            scale = (c['bg'] / np.sqrt(1.0 + 1e-5))[None]
            bias = (c['cb'] * scale[0] + c['bb'])[None]
            cw = c['cw'].astype(bf16)
            h = _conv_call(am1, x, ap1, cw[:, :, 0].T, cw[:, :, 1].T,
                           cw[:, :, 2].T, scale, bias)
            hb = h.reshape(B, L, D_MODEL)
            hp = jnp.pad(hb, ((0, 0), (1, 1), (0, 0)),
                         constant_values=-jnp.inf)
            P = (L - 1) // 2 + 1
            ca = hp[:, 0::2][:, :P].reshape(B * P, D_MODEL)
            aa = hp[:, 1::2][:, :P].reshape(B * P, D_MODEL)
            ba = hp[:, 2::2][:, :P].reshape(B * P, D_MODEL)
            x = _pool_call(aa, ba, ca)
            L = P

    z = _final_call(x, params['norm_g'][None], params['norm_b'][None])
    zf = z.reshape(B, L * D_MODEL)
    return _proj_call(zf, params['proj_w'].T.astype(bf16),
                      params['proj_b'][None])
